# Initial kernel scaffold; baseline (speedup 1.0000x reference)
import jax
import jax.numpy as jnp
from jax.experimental import pallas as pl


def kernel(scores, gold_target, transitions):
    raise NotImplementedError("write your pallas kernel here")


# speedup vs baseline: 19.7104x; 19.7104x over previous
import jax
import jax.numpy as jnp
from jax.experimental import pallas as pl

_K = 64
_START = 61
_END = 63


def _crf_fwd_kernel(scores_ref, gold_ref, trans_ref, out_ref):
    T = trans_ref[...]
    E = jnp.exp(T)
    L, B, _ = scores_ref.shape

    fs0 = jnp.broadcast_to(T[_START, :], (B, _K))

    def step(t, fs):
        s = scores_ref[pl.ds(t, 1)][0]
        a = fs + s
        m = jnp.max(a, axis=1, keepdims=True)
        u = jnp.exp(a - m)
        w = jax.lax.dot(
            u, E,
            precision=jax.lax.Precision.HIGHEST,
            preferred_element_type=jnp.float32,
        )
        return m + jnp.log(w)

    fs = jax.lax.fori_loop(0, L, step, fs0)
    forscores = jnp.sum(fs[:, _END])

    scores0 = jnp.sum(scores_ref[:, :, 0])
    gold = gold_ref[...]
    t0 = T[0, :]
    oh = gold[:, :, None] == jax.lax.broadcasted_iota(jnp.int32, (1, 1, _K), 2)
    tg_gather = jnp.sum(jnp.where(oh, t0[None, None, :], 0.0))
    tg_energy = B * T[0, _START] + scores0 + tg_gather

    loss = (forscores - tg_energy) / B
    out_ref[...] = jnp.full((8, 128), loss, dtype=jnp.float32)


def _run(scores_t, gold, trans, interpret=False):
    return pl.pallas_call(
        _crf_fwd_kernel,
        out_shape=jax.ShapeDtypeStruct((8, 128), jnp.float32),
        interpret=interpret,
    )(scores_t, gold, trans)


@jax.jit
def _kernel_jit(scores, gold_target, transitions):
    scores_t = jnp.transpose(scores, (1, 0, 2))
    return _run(scores_t, gold_target, transitions)[0, 0]


def kernel(scores, gold_target, transitions):
    return _kernel_jit(scores, gold_target, transitions)
